# Initial kernel scaffold; baseline (speedup 1.0000x reference)
import jax
import jax.numpy as jnp
from jax.experimental import pallas as pl


def kernel(x):
    raise NotImplementedError("write your pallas kernel here")


# speedup vs baseline: 3.4526x; 3.4526x over previous
import jax
import jax.numpy as jnp
from jax import lax
from jax.experimental import pallas as pl

K = 10
NEG = -1000000000.0
ROWS = 128
COLS = 32768
BLOCK_ROWS = 8


def _topk_mask_body(x_ref, o_ref):
    data = x_ref[...]
    r, n = data.shape
    col = lax.broadcasted_iota(jnp.int32, (r, n), 1)
    work = data
    for _ in range(K):
        m = jnp.max(work, axis=1, keepdims=True)
        cand = jnp.where(work == m, col, jnp.int32(n))
        idx = jnp.min(cand, axis=1, keepdims=True)
        work = jnp.where(col == idx, -jnp.inf, work)
    o_ref[...] = jnp.where(work == -jnp.inf, jnp.float32(NEG), work)


def kernel(x):
    return pl.pallas_call(
        _topk_mask_body,
        grid=(ROWS // BLOCK_ROWS,),
        in_specs=[pl.BlockSpec((BLOCK_ROWS, COLS), lambda i: (i, 0))],
        out_specs=pl.BlockSpec((BLOCK_ROWS, COLS), lambda i: (i, 0)),
        out_shape=jax.ShapeDtypeStruct((ROWS, COLS), jnp.float32),
    )(x)
